# baseline (speedup 1.0000x reference)
import jax
import jax.numpy as jnp
from jax import lax
from jax.experimental import pallas as pl
from jax.experimental.pallas import tpu as pltpu

D_MODEL = 768
HIDDEN = 2048
E = 64
T = 2048
K = 32
G = 8
NBLK = E // G
S = G * K


def _moe_body(x_ref, gw_ref, wk_ref, bk_ref, wv_ref, bv_ref, out_ref,
              s_scr, i_scr, sf_scr, if_scr, oacc_scr, v_scr):
    e = pl.program_id(0)

    @pl.when(e == 0)
    def _router():
        logits = jnp.dot(x_ref[...], gw_ref[...],
                         preferred_element_type=jnp.float32)
        m = jnp.max(logits, axis=1, keepdims=True)
        p = jnp.exp(logits - m)
        probs = p / jnp.sum(p, axis=1, keepdims=True)
        v_scr[...] = jnp.transpose(probs)

    @pl.when(e < NBLK)
    def _topk_block():
        base = e * G
        vals = v_scr[pl.ds(base, G), :]
        iota1 = lax.broadcasted_iota(jnp.int32, (G, T), 1)
        for i in range(K):
            mx = jnp.max(vals, axis=1, keepdims=True)
            am = jnp.min(jnp.where(vals == mx, iota1, T), axis=1, keepdims=True)
            s_scr[pl.ds(base, G), pl.ds(i, 1)] = mx
            i_scr[pl.ds(base, G), pl.ds(i, 1)] = am
            vals = jnp.where(iota1 == am, -jnp.inf, vals)
        for j in range(G):
            sf_scr[pl.ds(e, 1), :, pl.ds(j * K, K)] = (
                s_scr[pl.ds(base + j, 1), :].reshape(1, 1, K))
            if_scr[pl.ds(e, 1), :, pl.ds(j * K, K)] = (
                i_scr[pl.ds(base + j, 1), :].reshape(1, 1, K))

    idx_row = i_scr[pl.ds(e, 1), :]
    iota_t = lax.broadcasted_iota(jnp.int32, (T, K), 0)
    onehot = (iota_t == idx_row).astype(jnp.float32)
    g = lax.dot_general(onehot, x_ref[...], (((0,), (0,)), ((), ())),
                        preferred_element_type=jnp.float32)
    h = jax.nn.gelu(jnp.dot(g, wk_ref[0], preferred_element_type=jnp.float32)
                    + bk_ref[0])
    o = jnp.dot(h, wv_ref[0], preferred_element_type=jnp.float32) + bv_ref[0]
    oacc_scr[pl.ds(lax.rem(e, G) * K, K), :] = o

    @pl.when(lax.rem(e, G) == G - 1)
    def _flush():
        b = lax.div(e, G)
        idx_flat = if_scr[pl.ds(b, 1)].reshape(1, S)
        sc_flat = sf_scr[pl.ds(b, 1)].reshape(1, S)
        iota_s = lax.broadcasted_iota(jnp.int32, (T, S), 0)
        oh = (iota_s == idx_flat).astype(jnp.float32) * sc_flat
        contrib = jnp.dot(oh, oacc_scr[...], preferred_element_type=jnp.float32)

        @pl.when(b == 0)
        def _():
            out_ref[...] = contrib

        @pl.when(b != 0)
        def _():
            out_ref[...] += contrib


@jax.jit
def kernel(x, gate_W, Wk, bk, Wv, bv):
    b, s, d = x.shape
    x2d = x.reshape(T, D_MODEL)
    bk3 = bk.reshape(E, 1, HIDDEN)
    bv3 = bv.reshape(E, 1, D_MODEL)

    out = pl.pallas_call(
        _moe_body,
        grid=(E,),
        in_specs=[
            pl.BlockSpec((T, D_MODEL), lambda e: (0, 0)),
            pl.BlockSpec((D_MODEL, E), lambda e: (0, 0)),
            pl.BlockSpec((1, D_MODEL, HIDDEN), lambda e: (e, 0, 0)),
            pl.BlockSpec((1, 1, HIDDEN), lambda e: (e, 0, 0)),
            pl.BlockSpec((1, HIDDEN, D_MODEL), lambda e: (e, 0, 0)),
            pl.BlockSpec((1, 1, D_MODEL), lambda e: (e, 0, 0)),
        ],
        out_specs=pl.BlockSpec((T, D_MODEL), lambda e: (0, 0)),
        out_shape=jax.ShapeDtypeStruct((T, D_MODEL), jnp.float32),
        scratch_shapes=[
            pltpu.VMEM((E, K), jnp.float32),
            pltpu.VMEM((E, K), jnp.int32),
            pltpu.VMEM((NBLK, 1, S), jnp.float32),
            pltpu.VMEM((NBLK, 1, S), jnp.int32),
            pltpu.VMEM((S, D_MODEL), jnp.float32),
            pltpu.VMEM((E, T), jnp.float32),
        ],
    )(x2d, gate_W, Wk, bk3, Wv, bv3)

    return out.reshape(b, s, d)


# speedup vs baseline: 1.1737x; 1.1737x over previous
import jax
import jax.numpy as jnp
from jax import lax
from jax.experimental import pallas as pl
from jax.experimental.pallas import tpu as pltpu

D_MODEL = 768
HIDDEN = 2048
E = 64
T = 2048
K = 32
G = 16
NBLK = E // G
S = G * K


def _moe_body(x_ref, gw_ref, wk_ref, bk_ref, wv_ref, bv_ref, out_ref,
              s_scr, i_scr, sf_scr, if_scr, oacc_scr):
    e = pl.program_id(0)

    @pl.when(e == 0)
    def _router_topk():
        logits = jnp.dot(x_ref[...], gw_ref[...],
                         preferred_element_type=jnp.float32)
        m = jnp.max(logits, axis=1, keepdims=True)
        p = jnp.exp(logits - m)
        probs = p / jnp.sum(p, axis=1, keepdims=True)
        vals = jnp.transpose(probs)
        iota1 = lax.broadcasted_iota(jnp.int32, (E, T), 1)
        for i in range(K):
            mx = jnp.max(vals, axis=1, keepdims=True)
            am = jnp.min(jnp.where(vals == mx, iota1, T), axis=1, keepdims=True)
            s_scr[:, pl.ds(i, 1)] = mx
            i_scr[:, pl.ds(i, 1)] = am
            vals = jnp.where(iota1 == am, -jnp.inf, vals)
        for ee in range(E):
            sf_scr[ee // G, :, pl.ds((ee % G) * K, K)] = s_scr[pl.ds(ee, 1), :]
            if_scr[ee // G, :, pl.ds((ee % G) * K, K)] = i_scr[pl.ds(ee, 1), :]

    idx_row = i_scr[pl.ds(e, 1), :]
    iota_t = lax.broadcasted_iota(jnp.int32, (T, K), 0)
    onehot = (iota_t == idx_row).astype(jnp.float32)
    g = lax.dot_general(onehot, x_ref[...], (((0,), (0,)), ((), ())),
                        preferred_element_type=jnp.float32)
    h = jax.nn.gelu(jnp.dot(g, wk_ref[0], preferred_element_type=jnp.float32)
                    + bk_ref[0])
    o = jnp.dot(h, wv_ref[0], preferred_element_type=jnp.float32) + bv_ref[0]
    oacc_scr[pl.ds(lax.rem(e, G) * K, K), :] = o.astype(jnp.bfloat16)

    @pl.when(lax.rem(e, G) == G - 1)
    def _flush():
        b = lax.div(e, G)
        idx_flat = if_scr[pl.ds(b, 1)].reshape(1, S)
        sc_flat = sf_scr[pl.ds(b, 1)].reshape(1, S)
        iota_s = lax.broadcasted_iota(jnp.int32, (T, S), 0)
        oh = jnp.where(iota_s == idx_flat, sc_flat, 0.0).astype(jnp.bfloat16)
        contrib = jnp.dot(oh, oacc_scr[...], preferred_element_type=jnp.float32)

        @pl.when(b == 0)
        def _():
            out_ref[...] = contrib

        @pl.when(b != 0)
        def _():
            out_ref[...] += contrib


@jax.jit
def kernel(x, gate_W, Wk, bk, Wv, bv):
    b, s, d = x.shape
    x2d = x.reshape(T, D_MODEL)
    bk3 = bk.reshape(E, 1, HIDDEN)
    bv3 = bv.reshape(E, 1, D_MODEL)

    out = pl.pallas_call(
        _moe_body,
        grid=(E,),
        in_specs=[
            pl.BlockSpec((T, D_MODEL), lambda e: (0, 0)),
            pl.BlockSpec((D_MODEL, E), lambda e: (0, 0)),
            pl.BlockSpec((1, D_MODEL, HIDDEN), lambda e: (e, 0, 0)),
            pl.BlockSpec((1, 1, HIDDEN), lambda e: (e, 0, 0)),
            pl.BlockSpec((1, HIDDEN, D_MODEL), lambda e: (e, 0, 0)),
            pl.BlockSpec((1, 1, D_MODEL), lambda e: (e, 0, 0)),
        ],
        out_specs=pl.BlockSpec((T, D_MODEL), lambda e: (0, 0)),
        out_shape=jax.ShapeDtypeStruct((T, D_MODEL), jnp.float32),
        scratch_shapes=[
            pltpu.VMEM((E, K), jnp.float32),
            pltpu.VMEM((E, K), jnp.int32),
            pltpu.VMEM((NBLK, 1, S), jnp.float32),
            pltpu.VMEM((NBLK, 1, S), jnp.int32),
            pltpu.VMEM((S, D_MODEL), jnp.bfloat16),
        ],
    )(x2d, gate_W, Wk, bk3, Wv, bv3)

    return out.reshape(b, s, d)
